# baseline (speedup 1.0000x reference)
import functools

import jax
import jax.numpy as jnp
import numpy as np
from jax import lax
from jax.experimental import pallas as pl
from jax.experimental.pallas import tpu as pltpu
from jax.experimental.pallas import tpu_sc as plsc

_N = 10000
_E = 320000
_D = 128
_ED = 16

_NC = 2
_NS = 16
_NW = _NC * _NS

_C = 80
_PER_W = _E // _NW
_CHUNKS = _PER_W // _C
_RX = 3
_RI = 4
_RPT = 624
_REXTRA = _N - _RPT * _NS

_LOCOL = np.array([32 * j + k for j in range(_D // 32) for k in range(16)])
_HICOL = _LOCOL + 16


def _sc_aggregate(x, e, sd):
    mesh = plsc.VectorSubcoreMesh(core_axis_name="c", subcore_axis_name="s",
                                  num_cores=_NC, num_subcores=_NS)

    @functools.partial(
        pl.kernel,
        out_type=jax.ShapeDtypeStruct((_NC, _N, _D), jnp.float32),
        mesh=mesh,
        scratch_types=[
            pltpu.VMEM((2 * _RI, _C), jnp.int32),
            pltpu.VMEM((_RX, _C, _D), jnp.float32),
            pltpu.VMEM((_RX, _C // 2, _D), jnp.int32),
            pltpu.VMEM_SHARED((_N, _D), jnp.float32),
            [pltpu.SemaphoreType.DMA for _ in range(_RI + 3 * _RX)],
        ],
    )
    def body(x_hbm, e_hbm, sd_hbm, out_hbm, idx, xrows, erows, acc, sems):
        isem = sems[0:_RI]
        gsem = sems[_RI:_RI + _RX]
        lsem = sems[_RI + _RX:_RI + 2 * _RX]
        ssem = sems[_RI + 2 * _RX:_RI + 3 * _RX]
        c = lax.axis_index("c")
        s = lax.axis_index("s")
        w = c * _NS + s
        ebase = w * _PER_W

        zero = jnp.zeros((16,), jnp.float32)

        def zrow(r, _):
            for j in range(_D // 16):
                xrows[0, r, pl.ds(j * 16, 16)] = zero
            return 0

        lax.fori_loop(0, _C, zrow, 0)
        for k in range(_RPT // _C):
            r0 = pl.multiple_of(s * _RPT + k * _C, 8)
            pltpu.sync_copy(xrows.at[0], acc.at[pl.ds(r0, _C)])
        rem = _RPT - _RPT // _C * _C
        r0 = pl.multiple_of(s * _RPT + _RPT // _C * _C, 8)
        pltpu.sync_copy(xrows.at[0].at[pl.ds(0, rem)], acc.at[pl.ds(r0, rem)])

        @pl.when(s == 0)
        def _():
            pltpu.sync_copy(xrows.at[0].at[pl.ds(0, _REXTRA)],
                            acc.at[pl.ds(_RPT * _NS, _REXTRA)])

        plsc.subcore_barrier()

        def idesc(i, q):
            return pltpu.make_async_copy(sd_hbm.at[w, i], idx.at[pl.ds(2 * q, 2)],
                                         isem[q])

        def gdesc(i, b, q):
            return pltpu.make_async_copy(x_hbm.at[idx.at[2 * q]], xrows.at[b], gsem[b])

        def ldesc(i, b):
            eb = pl.multiple_of((ebase + i * _C) // 2, 8)
            return pltpu.make_async_copy(e_hbm.at[pl.ds(eb, _C // 2)], erows.at[b],
                                         lsem[b])

        def sdesc(i, b, q):
            return pltpu.make_async_copy(xrows.at[b], acc.at[idx.at[2 * q + 1]],
                                         ssem[b])

        himask = jnp.full((16,), -65536, jnp.int32)

        def compute(b):
            def rowpair(rp, _):
                for h in range(2):
                    r = rp + h * (_C // 2)
                    for j in range(_D // 32):
                        ei = erows[b, rp, pl.ds(h * 64 + j * 16, 16)]
                        elo = jax.lax.bitcast_convert_type(ei << 16, jnp.float32)
                        ehi = jax.lax.bitcast_convert_type(ei & himask, jnp.float32)
                        slo = pl.ds(j * 32, 16)
                        shi = pl.ds(j * 32 + 16, 16)
                        xrows[b, r, slo] = jnp.maximum(xrows[b, r, slo] + elo, 0.0)
                        xrows[b, r, shi] = jnp.maximum(xrows[b, r, shi] + ehi, 0.0)
                return 0

            lax.fori_loop(0, _C // 2, rowpair, 0, unroll=4)

        def half(i, t, guard_lo=True, do_next=True, do_next_idx=True):
            b = t % _RX
            q = t % _RI
            jb = (t + 1) % _RX
            jq = (t + 1) % _RI
            kq = (t + 2) % _RI
            if guard_lo:

                @pl.when(i >= 2)
                def _():
                    sdesc(i - 2, jb, kq).wait()
            else:
                sdesc(i - 2, jb, kq).wait()
            if do_next:
                idesc(i + 1, jq).wait()
                gdesc(i + 1, jb, jq).start()
                ldesc(i + 1, jb).start()
            if do_next_idx:
                idesc(i + 2, kq).start()
            gdesc(i, b, q).wait()
            ldesc(i, b).wait()
            compute(b)
            sdesc(i, b, q).start(add=True)

        idesc(0, 0).start()
        idesc(1, 1).start()
        idesc(0, 0).wait()
        gdesc(0, 0, 0).start()
        ldesc(0, 0).start()

        _G = 12
        _NG = (_CHUNKS - 5) // _G

        @pl.loop(0, _NG)
        def _(g):
            i0 = g * _G
            for t in range(_G):
                half(i0 + t, t)

        for i in range(_NG * _G, _CHUNKS):
            half(i, i % _G, guard_lo=False,
                 do_next=i + 1 < _CHUNKS, do_next_idx=i + 2 < _CHUNKS)

        sdesc(_CHUNKS - 2, (_CHUNKS - 2) % _RX, (_CHUNKS - 2) % _RI).wait()
        sdesc(_CHUNKS - 1, (_CHUNKS - 1) % _RX, (_CHUNKS - 1) % _RI).wait()
        plsc.subcore_barrier()

        nro = _RPT // _C
        for k in range(nro):
            b = k % _RX
            if k >= _RX:
                pltpu.make_async_copy(xrows.at[b], out_hbm.at[c, pl.ds(0, _C)],
                                      gsem[b]).wait()
            r0 = pl.multiple_of(s * _RPT + k * _C, 8)
            pltpu.sync_copy(acc.at[pl.ds(r0, _C)], xrows.at[b])
            pltpu.make_async_copy(xrows.at[b], out_hbm.at[c, pl.ds(r0, _C)],
                                  gsem[b]).start()
        for b in range(_RX):
            pltpu.make_async_copy(xrows.at[b], out_hbm.at[c, pl.ds(0, _C)],
                                  gsem[b]).wait()
        rem = _RPT - nro * _C
        r0 = pl.multiple_of(s * _RPT + nro * _C, 8)
        pltpu.sync_copy(acc.at[pl.ds(r0, rem)], xrows.at[0].at[pl.ds(0, rem)])
        pltpu.sync_copy(xrows.at[0].at[pl.ds(0, rem)], out_hbm.at[c, pl.ds(r0, rem)])

        @pl.when(s == 0)
        def _():
            pltpu.sync_copy(acc.at[pl.ds(_RPT * _NS, _REXTRA)],
                            xrows.at[1].at[pl.ds(0, _REXTRA)])
            pltpu.sync_copy(xrows.at[1].at[pl.ds(0, _REXTRA)],
                            out_hbm.at[c, pl.ds(_RPT * _NS, _REXTRA)])

    return body(x, e, sd)


def _dot(a, b):
    return jax.lax.dot_general(a, b, (((1,), (0,)), ((), ())),
                               preferred_element_type=jnp.float32)


_EBLK = 16000


def _edge_lin2(ef, W1lo, b1lo, W1hi, b1hi):
    def pack(lo, hi):
        lo16 = jax.lax.bitcast_convert_type(lo.astype(jnp.bfloat16), jnp.uint16)
        hi16 = jax.lax.bitcast_convert_type(hi.astype(jnp.bfloat16), jnp.uint16)
        packed = lo16.astype(jnp.int32) | (hi16.astype(jnp.int32) << 16)
        p4 = packed.reshape(_EBLK // _C, 2, _C // 2, _D // 2)
        return jnp.concatenate([p4[:, 0], p4[:, 1]], axis=-1).reshape(
            _EBLK // 2, _D)

    def body(ef_ref, w1l_ref, b1l_ref, w1h_ref, b1h_ref, o1_ref):
        a = ef_ref[...]
        o1_ref[...] = pack(_dot(a, w1l_ref[...]) + b1l_ref[...],
                           _dot(a, w1h_ref[...]) + b1h_ref[...])

    wspec = pl.BlockSpec((_ED, _D // 2), lambda i: (0, 0))
    bspec = pl.BlockSpec((1, _D // 2), lambda i: (0, 0))
    return pl.pallas_call(
        body,
        grid=(_E // _EBLK,),
        in_specs=[
            pl.BlockSpec((_EBLK, _ED), lambda i: (i, 0)),
            wspec, bspec, wspec, bspec,
        ],
        out_specs=pl.BlockSpec((_EBLK // 2, _D), lambda i: (i, 0)),
        out_shape=jax.ShapeDtypeStruct((_E // 2, _D), jnp.int32),
    )(ef, W1lo, b1lo.reshape(1, _D // 2), W1hi, b1hi.reshape(1, _D // 2))


_NBLK = 2000


def _node_mlp(h, part, Wa, ba, Wb, bb):
    def body(h_ref, p_ref, wa_ref, ba_ref, wb_ref, bb_ref, o_ref):
        h0 = h_ref[...] + p_ref[0] + p_ref[1]
        t = jnp.maximum(_dot(h0, wa_ref[...]) + ba_ref[...], 0.0)
        o_ref[...] = jnp.tanh(_dot(t, wb_ref[...]) + bb_ref[...])

    return pl.pallas_call(
        body,
        grid=(_N // _NBLK,),
        in_specs=[
            pl.BlockSpec((_NBLK, _D), lambda i: (i, 0)),
            pl.BlockSpec((_NC, _NBLK, _D), lambda i: (0, i, 0)),
            pl.BlockSpec((_D, _D), lambda i: (0, 0)),
            pl.BlockSpec((1, _D), lambda i: (0, 0)),
            pl.BlockSpec((_D, _D), lambda i: (0, 0)),
            pl.BlockSpec((1, _D), lambda i: (0, 0)),
        ],
        out_specs=pl.BlockSpec((_NBLK, _D), lambda i: (i, 0)),
        out_shape=jax.ShapeDtypeStruct((_N, _D), jnp.float32),
    )(h, part, Wa, ba.reshape(1, _D), Wb, bb.reshape(1, _D))


def _node_mlp_fc(h, part, Wa, ba, Wb, bb, Wf1, bf1, Wf2, bf2):
    def body(h_ref, p_ref, wa_ref, ba_ref, wb_ref, bb_ref,
             wf1_ref, bf1_ref, wf2_ref, bf2_ref, o_ref):
        h0 = h_ref[...] + p_ref[0] + p_ref[1]
        t = jnp.maximum(_dot(h0, wa_ref[...]) + ba_ref[...], 0.0)
        h2 = jnp.tanh(_dot(t, wb_ref[...]) + bb_ref[...])
        h3 = jnp.tanh(_dot(h2, wf1_ref[...]) + bf1_ref[...])
        o_ref[...] = _dot(h3, wf2_ref[...]) + bf2_ref[...]

    wspec = pl.BlockSpec((_D, _D), lambda i: (0, 0))
    bspec = pl.BlockSpec((1, _D), lambda i: (0, 0))
    return pl.pallas_call(
        body,
        grid=(_N // _NBLK,),
        in_specs=[
            pl.BlockSpec((_NBLK, _D), lambda i: (i, 0)),
            pl.BlockSpec((_NC, _NBLK, _D), lambda i: (0, i, 0)),
            wspec, bspec, wspec, bspec, wspec, bspec, wspec, bspec,
        ],
        out_specs=pl.BlockSpec((_NBLK, _D), lambda i: (i, 0)),
        out_shape=jax.ShapeDtypeStruct((_N, _D), jnp.float32),
    )(h, part, Wa, ba.reshape(1, _D), Wb, bb.reshape(1, _D),
      Wf1, bf1.reshape(1, _D), Wf2, bf2.reshape(1, _D))


def kernel(x, edge_index, edge_feats,
           We1, be1, W1a, b1a, W1b, b1b,
           We2, be2, W2a, b2a, W2b, b2b,
           Wf1, bf1, Wf2, bf2):
    sd = jnp.stack([edge_index[0].reshape(_NW, _CHUNKS, _C),
                    edge_index[1].reshape(_NW, _CHUNKS, _C)], axis=2)
    lo = jnp.asarray(_LOCOL)
    hi = jnp.asarray(_HICOL)
    e1 = _edge_lin2(edge_feats, We1[:, lo], be1[lo], We1[:, hi], be1[hi])
    e2 = _edge_lin2(edge_feats, We2[:, lo], be2[lo], We2[:, hi], be2[hi])
    p1 = _sc_aggregate(x, e1, sd)
    h1 = _node_mlp(x, p1, W1a, b1a, W1b, b1b)
    p2 = _sc_aggregate(h1, e2, sd)
    return _node_mlp_fc(h1, p2, W2a, b2a, W2b, b2b, Wf1, bf1, Wf2, bf2)


# speedup vs baseline: 1.6410x; 1.6410x over previous
import functools

import jax
import jax.numpy as jnp
import numpy as np
from jax import lax
from jax.experimental import pallas as pl
from jax.experimental.pallas import tpu as pltpu
from jax.experimental.pallas import tpu_sc as plsc

_N = 10000
_E = 320000
_D = 128
_ED = 16

_NC = 2
_NS = 16
_NW = _NC * _NS

_C = 80
_PER_W = _E // _NW
_CHUNKS = _PER_W // _C
_RX = 3
_RI = 4
_RPT = 624
_REXTRA = _N - _RPT * _NS

_LOCOL = np.array([32 * j + k for j in range(_D // 32) for k in range(16)])
_HICOL = _LOCOL + 16


def _sc_aggregate(x, e, sd):
    mesh = plsc.VectorSubcoreMesh(core_axis_name="c", subcore_axis_name="s",
                                  num_cores=_NC, num_subcores=_NS)

    @functools.partial(
        pl.kernel,
        out_type=jax.ShapeDtypeStruct((_NC, _N, _D), jnp.float32),
        mesh=mesh,
        scratch_types=[
            pltpu.VMEM((2 * _RI, _C), jnp.int32),
            pltpu.VMEM((_RX, _C, _D), jnp.float32),
            pltpu.VMEM((_RX, _C // 2, _D), jnp.int32),
            pltpu.VMEM_SHARED((_N, _D), jnp.float32),
            [pltpu.SemaphoreType.DMA for _ in range(_RI + 3 * _RX)],
        ],
    )
    def body(x_hbm, e_hbm, sd_hbm, out_hbm, idx, xrows, erows, acc, sems):
        isem = sems[0:_RI]
        gsem = sems[_RI:_RI + _RX]
        lsem = sems[_RI + _RX:_RI + 2 * _RX]
        ssem = sems[_RI + 2 * _RX:_RI + 3 * _RX]
        c = lax.axis_index("c")
        s = lax.axis_index("s")
        w = c * _NS + s
        ebase = w * _PER_W

        zero = jnp.zeros((16,), jnp.float32)

        def zrow(r, _):
            for j in range(_D // 16):
                xrows[0, r, pl.ds(j * 16, 16)] = zero
            return 0

        lax.fori_loop(0, _C, zrow, 0)
        for k in range(_RPT // _C):
            r0 = pl.multiple_of(s * _RPT + k * _C, 8)
            pltpu.sync_copy(xrows.at[0], acc.at[pl.ds(r0, _C)])
        rem = _RPT - _RPT // _C * _C
        r0 = pl.multiple_of(s * _RPT + _RPT // _C * _C, 8)
        pltpu.sync_copy(xrows.at[0].at[pl.ds(0, rem)], acc.at[pl.ds(r0, rem)])

        @pl.when(s == 0)
        def _():
            pltpu.sync_copy(xrows.at[0].at[pl.ds(0, _REXTRA)],
                            acc.at[pl.ds(_RPT * _NS, _REXTRA)])

        plsc.subcore_barrier()

        def idesc(i, q):
            return pltpu.make_async_copy(sd_hbm.at[w, i], idx.at[pl.ds(2 * q, 2)],
                                         isem[q])

        def gdesc(i, b, q):
            return pltpu.make_async_copy(x_hbm.at[idx.at[2 * q]], xrows.at[b], gsem[b])

        def ldesc(i, b):
            eb = pl.multiple_of((ebase + i * _C) // 2, 8)
            return pltpu.make_async_copy(e_hbm.at[pl.ds(eb, _C // 2)], erows.at[b],
                                         lsem[b])

        def sdesc(i, b, q):
            return pltpu.make_async_copy(xrows.at[b], acc.at[idx.at[2 * q + 1]],
                                         ssem[b])

        himask = jnp.full((16,), -65536, jnp.int32)

        def compute(b):
            def rowpair(rp, _):
                for h in range(2):
                    r = rp + h * (_C // 2)
                    for j in range(_D // 32):
                        ei = erows[b, rp, pl.ds(h * 64 + j * 16, 16)]
                        elo = jax.lax.bitcast_convert_type(ei << 16, jnp.float32)
                        ehi = jax.lax.bitcast_convert_type(ei & himask, jnp.float32)
                        slo = pl.ds(j * 32, 16)
                        shi = pl.ds(j * 32 + 16, 16)
                        xrows[b, r, slo] = jnp.maximum(xrows[b, r, slo] + elo, 0.0)
                        xrows[b, r, shi] = jnp.maximum(xrows[b, r, shi] + ehi, 0.0)
                return 0

            lax.fori_loop(0, _C // 2, rowpair, 0)

        def half(i, t, guard_lo=True, do_next=True, do_next_idx=True):
            b = t % _RX
            q = t % _RI
            jb = (t + 1) % _RX
            jq = (t + 1) % _RI
            kq = (t + 2) % _RI
            if guard_lo:

                @pl.when(i >= 2)
                def _():
                    sdesc(i - 2, jb, kq).wait()
            else:
                sdesc(i - 2, jb, kq).wait()
            if do_next:
                idesc(i + 1, jq).wait()
                gdesc(i + 1, jb, jq).start()
                ldesc(i + 1, jb).start()
            if do_next_idx:
                idesc(i + 2, kq).start()
            gdesc(i, b, q).wait()
            ldesc(i, b).wait()
            compute(b)
            sdesc(i, b, q).start(add=True)

        idesc(0, 0).start()
        idesc(1, 1).start()
        idesc(0, 0).wait()
        gdesc(0, 0, 0).start()
        ldesc(0, 0).start()

        _G = 12
        _NG = (_CHUNKS - 5) // _G

        @pl.loop(0, _NG)
        def _(g):
            i0 = g * _G
            for t in range(_G):
                half(i0 + t, t)

        for i in range(_NG * _G, _CHUNKS):
            half(i, i % _G, guard_lo=False,
                 do_next=i + 1 < _CHUNKS, do_next_idx=i + 2 < _CHUNKS)

        sdesc(_CHUNKS - 2, (_CHUNKS - 2) % _RX, (_CHUNKS - 2) % _RI).wait()
        sdesc(_CHUNKS - 1, (_CHUNKS - 1) % _RX, (_CHUNKS - 1) % _RI).wait()
        plsc.subcore_barrier()

        nro = _RPT // _C
        for k in range(nro):
            b = k % _RX
            if k >= _RX:
                pltpu.make_async_copy(xrows.at[b], out_hbm.at[c, pl.ds(0, _C)],
                                      gsem[b]).wait()
            r0 = pl.multiple_of(s * _RPT + k * _C, 8)
            pltpu.sync_copy(acc.at[pl.ds(r0, _C)], xrows.at[b])
            pltpu.make_async_copy(xrows.at[b], out_hbm.at[c, pl.ds(r0, _C)],
                                  gsem[b]).start()
        for b in range(_RX):
            pltpu.make_async_copy(xrows.at[b], out_hbm.at[c, pl.ds(0, _C)],
                                  gsem[b]).wait()
        rem = _RPT - nro * _C
        r0 = pl.multiple_of(s * _RPT + nro * _C, 8)
        pltpu.sync_copy(acc.at[pl.ds(r0, rem)], xrows.at[0].at[pl.ds(0, rem)])
        pltpu.sync_copy(xrows.at[0].at[pl.ds(0, rem)], out_hbm.at[c, pl.ds(r0, rem)])

        @pl.when(s == 0)
        def _():
            pltpu.sync_copy(acc.at[pl.ds(_RPT * _NS, _REXTRA)],
                            xrows.at[1].at[pl.ds(0, _REXTRA)])
            pltpu.sync_copy(xrows.at[1].at[pl.ds(0, _REXTRA)],
                            out_hbm.at[c, pl.ds(_RPT * _NS, _REXTRA)])

    return body(x, e, sd)


def _dot(a, b):
    return jax.lax.dot_general(a, b, (((1,), (0,)), ((), ())),
                               preferred_element_type=jnp.float32)


_EBLK = 16000


def _edge_lin2(ef, W1lo, b1lo, W1hi, b1hi):
    def pack(lo, hi):
        lo16 = jax.lax.bitcast_convert_type(lo.astype(jnp.bfloat16), jnp.uint16)
        hi16 = jax.lax.bitcast_convert_type(hi.astype(jnp.bfloat16), jnp.uint16)
        packed = lo16.astype(jnp.int32) | (hi16.astype(jnp.int32) << 16)
        p4 = packed.reshape(_EBLK // _C, 2, _C // 2, _D // 2)
        return jnp.concatenate([p4[:, 0], p4[:, 1]], axis=-1).reshape(
            _EBLK // 2, _D)

    def body(ef_ref, w1l_ref, b1l_ref, w1h_ref, b1h_ref, o1_ref):
        a = ef_ref[...]
        o1_ref[...] = pack(_dot(a, w1l_ref[...]) + b1l_ref[...],
                           _dot(a, w1h_ref[...]) + b1h_ref[...])

    wspec = pl.BlockSpec((_ED, _D // 2), lambda i: (0, 0))
    bspec = pl.BlockSpec((1, _D // 2), lambda i: (0, 0))
    return pl.pallas_call(
        body,
        grid=(_E // _EBLK,),
        in_specs=[
            pl.BlockSpec((_EBLK, _ED), lambda i: (i, 0)),
            wspec, bspec, wspec, bspec,
        ],
        out_specs=pl.BlockSpec((_EBLK // 2, _D), lambda i: (i, 0)),
        out_shape=jax.ShapeDtypeStruct((_E // 2, _D), jnp.int32),
    )(ef, W1lo, b1lo.reshape(1, _D // 2), W1hi, b1hi.reshape(1, _D // 2))


_NBLK = 2000


def _node_mlp(h, part, Wa, ba, Wb, bb):
    def body(h_ref, p_ref, wa_ref, ba_ref, wb_ref, bb_ref, o_ref):
        h0 = h_ref[...] + p_ref[0] + p_ref[1]
        t = jnp.maximum(_dot(h0, wa_ref[...]) + ba_ref[...], 0.0)
        o_ref[...] = jnp.tanh(_dot(t, wb_ref[...]) + bb_ref[...])

    return pl.pallas_call(
        body,
        grid=(_N // _NBLK,),
        in_specs=[
            pl.BlockSpec((_NBLK, _D), lambda i: (i, 0)),
            pl.BlockSpec((_NC, _NBLK, _D), lambda i: (0, i, 0)),
            pl.BlockSpec((_D, _D), lambda i: (0, 0)),
            pl.BlockSpec((1, _D), lambda i: (0, 0)),
            pl.BlockSpec((_D, _D), lambda i: (0, 0)),
            pl.BlockSpec((1, _D), lambda i: (0, 0)),
        ],
        out_specs=pl.BlockSpec((_NBLK, _D), lambda i: (i, 0)),
        out_shape=jax.ShapeDtypeStruct((_N, _D), jnp.float32),
    )(h, part, Wa, ba.reshape(1, _D), Wb, bb.reshape(1, _D))


def _node_mlp_fc(h, part, Wa, ba, Wb, bb, Wf1, bf1, Wf2, bf2):
    def body(h_ref, p_ref, wa_ref, ba_ref, wb_ref, bb_ref,
             wf1_ref, bf1_ref, wf2_ref, bf2_ref, o_ref):
        h0 = h_ref[...] + p_ref[0] + p_ref[1]
        t = jnp.maximum(_dot(h0, wa_ref[...]) + ba_ref[...], 0.0)
        h2 = jnp.tanh(_dot(t, wb_ref[...]) + bb_ref[...])
        h3 = jnp.tanh(_dot(h2, wf1_ref[...]) + bf1_ref[...])
        o_ref[...] = _dot(h3, wf2_ref[...]) + bf2_ref[...]

    wspec = pl.BlockSpec((_D, _D), lambda i: (0, 0))
    bspec = pl.BlockSpec((1, _D), lambda i: (0, 0))
    return pl.pallas_call(
        body,
        grid=(_N // _NBLK,),
        in_specs=[
            pl.BlockSpec((_NBLK, _D), lambda i: (i, 0)),
            pl.BlockSpec((_NC, _NBLK, _D), lambda i: (0, i, 0)),
            wspec, bspec, wspec, bspec, wspec, bspec, wspec, bspec,
        ],
        out_specs=pl.BlockSpec((_NBLK, _D), lambda i: (i, 0)),
        out_shape=jax.ShapeDtypeStruct((_N, _D), jnp.float32),
    )(h, part, Wa, ba.reshape(1, _D), Wb, bb.reshape(1, _D),
      Wf1, bf1.reshape(1, _D), Wf2, bf2.reshape(1, _D))


def kernel(x, edge_index, edge_feats,
           We1, be1, W1a, b1a, W1b, b1b,
           We2, be2, W2a, b2a, W2b, b2b,
           Wf1, bf1, Wf2, bf2):
    sd = jnp.stack([edge_index[0].reshape(_NW, _CHUNKS, _C),
                    edge_index[1].reshape(_NW, _CHUNKS, _C)], axis=2)
    lo = jnp.asarray(_LOCOL)
    hi = jnp.asarray(_HICOL)
    e1 = _edge_lin2(edge_feats, We1[:, lo], be1[lo], We1[:, hi], be1[hi])
    e2 = _edge_lin2(edge_feats, We2[:, lo], be2[lo], We2[:, hi], be2[hi])
    p1 = _sc_aggregate(x, e1, sd)
    h1 = _node_mlp(x, p1, W1a, b1a, W1b, b1b)
    p2 = _sc_aggregate(h1, e2, sd)
    return _node_mlp_fc(h1, p2, W2a, b2a, W2b, b2b, Wf1, bf1, Wf2, bf2)
